# Initial kernel scaffold; baseline (speedup 1.0000x reference)
import jax
import jax.numpy as jnp
from jax.experimental import pallas as pl


def kernel(input_tensor, slices_index, slice_len):
    raise NotImplementedError("write your pallas kernel here")


# speedup vs baseline: 2.4685x; 2.4685x over previous
import functools

import jax
import jax.numpy as jnp
from jax import lax
from jax.experimental import pallas as pl
from jax.experimental.pallas import tpu as pltpu
from jax.experimental.pallas import tpu_sc as plsc

NC = 2
NS = 16
LANES = 16
CH = 128


def _fuse_slice_sc(inp, starts, S, B, L, rows_total, per_w, n_ch):
    bshift = B.bit_length() - 1

    mesh = plsc.VectorSubcoreMesh(
        core_axis_name="c", subcore_axis_name="s",
        num_cores=NC, num_subcores=NS)

    @functools.partial(
        pl.kernel,
        out_type=jax.ShapeDtypeStruct((rows_total, L), jnp.float32),
        mesh=mesh,
        scratch_types=[
            pltpu.VMEM((S + LANES,), jnp.int32),
            pltpu.VMEM((CH, L), jnp.float32),
            pltpu.SemaphoreType.DMA,
        ],
    )
    def k(inp_hbm, starts_hbm, out_hbm, starts_v, rows_v, sem):
        wid = lax.axis_index("s") * NC + lax.axis_index("c")
        pltpu.sync_copy(starts_hbm, starts_v.at[pl.ds(0, S)])

        def body(g, carry):
            o0 = pl.multiple_of(wid * per_w + g * CH, CH)
            s = lax.shift_right_logical(o0, bshift)
            b0 = pl.multiple_of(o0 - lax.shift_left(s, bshift), CH)
            st = pl.multiple_of(starts_v[pl.ds(s, LANES)][0], L)
            pltpu.async_copy(
                inp_hbm.at[pl.ds(b0, CH), pl.ds(st, L)], rows_v, sem).wait()
            pltpu.sync_copy(rows_v, out_hbm.at[pl.ds(o0, CH)])
            return carry

        lax.fori_loop(0, n_ch, body, 0)

    return k(inp, starts)


def kernel(input_tensor, slices_index, slice_len):
    B, total = input_tensor.shape
    S = slices_index.shape[0]
    L = total // S
    zero_offset = jnp.asarray(slice_len, jnp.int32) - jnp.int32(L)
    starts = slices_index.astype(jnp.int32) + zero_offset

    rows_total = S * B
    per_w = rows_total // (NC * NS)
    n_ch = per_w // CH
    assert per_w * NC * NS == rows_total and n_ch * CH == per_w

    out = _fuse_slice_sc(input_tensor, starts, S, B, L, rows_total, per_w, n_ch)
    return out.reshape(S, B, L)


# speedup vs baseline: 3.5605x; 1.4424x over previous
import functools

import jax
import jax.numpy as jnp
from jax import lax
from jax.experimental import pallas as pl
from jax.experimental.pallas import tpu as pltpu
from jax.experimental.pallas import tpu_sc as plsc

NC = 2
NS = 16
LANES = 16
CH = 128
NB = 4


def _fuse_slice_sc(inp, starts, S, B, L, rows_total, per_w, n_ch):
    bshift = B.bit_length() - 1

    mesh = plsc.VectorSubcoreMesh(
        core_axis_name="c", subcore_axis_name="s",
        num_cores=NC, num_subcores=NS)

    @functools.partial(
        pl.kernel,
        out_type=jax.ShapeDtypeStruct((rows_total, L), jnp.float32),
        mesh=mesh,
        scratch_types=(
            [pltpu.VMEM((S + LANES,), jnp.int32)]
            + [pltpu.VMEM((CH, L), jnp.float32)] * NB
            + [pltpu.SemaphoreType.DMA] * (2 * NB)
        ),
    )
    def k(inp_hbm, starts_hbm, out_hbm, starts_v, *rest):
        bufs = rest[:NB]
        gsems = rest[NB:2 * NB]
        ssems = rest[2 * NB:]
        wid = lax.axis_index("s") * NC + lax.axis_index("c")
        pltpu.sync_copy(starts_hbm, starts_v.at[pl.ds(0, S)])
        base = wid * per_w

        def out_off(g):
            return pl.multiple_of(base + g * CH, CH)

        def start_gather(g, buf, gsem):
            o0 = out_off(g)
            s = lax.shift_right_logical(o0, bshift)
            b0 = pl.multiple_of(o0 - lax.shift_left(s, bshift), CH)
            st = pl.multiple_of(starts_v[pl.ds(s, LANES)][0], L)
            pltpu.async_copy(inp_hbm.at[pl.ds(b0, CH), pl.ds(st, L)], buf, gsem)

        def wait_gather(buf, gsem):
            pltpu.make_async_copy(inp_hbm.at[pl.ds(0, CH), pl.ds(0, L)], buf, gsem).wait()

        def wait_scatter(buf, ssem):
            pltpu.make_async_copy(buf, out_hbm.at[pl.ds(0, CH)], ssem).wait()

        for b in range(NB):
            start_gather(b, bufs[b], gsems[b])

        def body(i, carry):
            for b in range(NB):
                g = i * NB + b
                wait_gather(bufs[b], gsems[b])
                pltpu.async_copy(bufs[b], out_hbm.at[pl.ds(out_off(g), CH)], ssems[b])
                h = g + NB - 2
                b2 = (b + NB - 2) % NB

                @pl.when(jnp.logical_and(h >= NB, h < n_ch))
                def _():
                    wait_scatter(bufs[b2], ssems[b2])
                    start_gather(h, bufs[b2], gsems[b2])
            return carry

        lax.fori_loop(0, n_ch // NB, body, 0)
        for b in range(NB):
            wait_scatter(bufs[b], ssems[b])

    return k(inp, starts)


def kernel(input_tensor, slices_index, slice_len):
    B, total = input_tensor.shape
    S = slices_index.shape[0]
    L = total // S
    zero_offset = jnp.asarray(slice_len, jnp.int32) - jnp.int32(L)
    starts = slices_index.astype(jnp.int32) + zero_offset

    rows_total = S * B
    per_w = rows_total // (NC * NS)
    n_ch = per_w // CH
    assert per_w * NC * NS == rows_total and n_ch * CH == per_w

    out = _fuse_slice_sc(input_tensor, starts, S, B, L, rows_total, per_w, n_ch)
    return out.reshape(S, B, L)


# speedup vs baseline: 3.5912x; 1.0086x over previous
import functools

import jax
import jax.numpy as jnp
from jax import lax
from jax.experimental import pallas as pl
from jax.experimental.pallas import tpu as pltpu
from jax.experimental.pallas import tpu_sc as plsc

NC = 2
NS = 16
LANES = 16
CH = 64
NB = 8
SLACK = 3


def _fuse_slice_sc(inp, starts, S, B, L, rows_total, per_w, n_ch):
    bshift = B.bit_length() - 1

    mesh = plsc.VectorSubcoreMesh(
        core_axis_name="c", subcore_axis_name="s",
        num_cores=NC, num_subcores=NS)

    @functools.partial(
        pl.kernel,
        out_type=jax.ShapeDtypeStruct((rows_total, L), jnp.float32),
        mesh=mesh,
        scratch_types=(
            [pltpu.VMEM((S + LANES,), jnp.int32)]
            + [pltpu.VMEM((CH, L), jnp.float32)] * NB
            + [pltpu.SemaphoreType.DMA] * (2 * NB)
        ),
    )
    def k(inp_hbm, starts_hbm, out_hbm, starts_v, *rest):
        bufs = rest[:NB]
        gsems = rest[NB:2 * NB]
        ssems = rest[2 * NB:]
        wid = lax.axis_index("s") * NC + lax.axis_index("c")
        pltpu.sync_copy(starts_hbm, starts_v.at[pl.ds(0, S)])
        base = wid * per_w

        def out_off(g):
            return pl.multiple_of(base + g * CH, CH)

        def start_gather(g, buf, gsem):
            o0 = out_off(g)
            s = lax.shift_right_logical(o0, bshift)
            b0 = pl.multiple_of(o0 - lax.shift_left(s, bshift), CH)
            st = pl.multiple_of(starts_v[pl.ds(s, LANES)][0], L)
            pltpu.async_copy(inp_hbm.at[pl.ds(b0, CH), pl.ds(st, L)], buf, gsem)

        def wait_gather(buf, gsem):
            pltpu.make_async_copy(inp_hbm.at[pl.ds(0, CH), pl.ds(0, L)], buf, gsem).wait()

        def wait_scatter(buf, ssem):
            pltpu.make_async_copy(buf, out_hbm.at[pl.ds(0, CH)], ssem).wait()

        for b in range(NB):
            start_gather(b, bufs[b], gsems[b])

        def body(i, carry):
            for b in range(NB):
                g = i * NB + b
                wait_gather(bufs[b], gsems[b])
                pltpu.async_copy(bufs[b], out_hbm.at[pl.ds(out_off(g), CH)], ssems[b])
                h = g + NB - SLACK
                b2 = (b + NB - SLACK) % NB

                @pl.when(jnp.logical_and(h >= NB, h < n_ch))
                def _():
                    wait_scatter(bufs[b2], ssems[b2])
                    start_gather(h, bufs[b2], gsems[b2])
            return carry

        lax.fori_loop(0, n_ch // NB, body, 0)
        for b in range(NB):
            wait_scatter(bufs[b], ssems[b])

    return k(inp, starts)


def kernel(input_tensor, slices_index, slice_len):
    B, total = input_tensor.shape
    S = slices_index.shape[0]
    L = total // S
    zero_offset = jnp.asarray(slice_len, jnp.int32) - jnp.int32(L)
    starts = slices_index.astype(jnp.int32) + zero_offset

    rows_total = S * B
    per_w = rows_total // (NC * NS)
    n_ch = per_w // CH
    assert per_w * NC * NS == rows_total and n_ch * CH == per_w

    out = _fuse_slice_sc(input_tensor, starts, S, B, L, rows_total, per_w, n_ch)
    return out.reshape(S, B, L)
